# baseline (speedup 1.0000x reference)
import functools

import jax
import jax.numpy as jnp
from jax import lax
from jax.experimental import pallas as pl
from jax.experimental.pallas import tpu as pltpu
from jax.experimental.pallas import tpu_sc as plsc

_B = 4096
_S = 200
_D = 64
_R = 4
_OUT_S = _R + _S
_NW = 32
_SEQ_PER_W = _B // _NW
_LANES = 16
_C0 = 128
_C1 = _S - _C0


def _make_kernel():
    mesh = plsc.VectorSubcoreMesh(core_axis_name="c", subcore_axis_name="s")

    @functools.partial(
        pl.kernel,
        mesh=mesh,
        compiler_params=pltpu.CompilerParams(use_tc_tiling_on_sc=False),
        out_type=jax.ShapeDtypeStruct((_B * _S // 2, 2 * _D), jnp.float32),
        scratch_types=[
            pltpu.VMEM((_S // 2, 2 * _D), jnp.float32),
            pltpu.VMEM((256,), jnp.int32),
            pltpu.VMEM((256,), jnp.int32),
            pltpu.VMEM((_S, _D), jnp.float32),
            pltpu.VMEM((_S, _D), jnp.float32),
            pltpu.VMEM((_S // 2, 2 * _D), jnp.float32),
            pltpu.VMEM((_S // 2, 2 * _D), jnp.float32),
            pltpu.SemaphoreType.DMA,
            pltpu.SemaphoreType.DMA,
            pltpu.SemaphoreType.DMA,
            pltpu.SemaphoreType.DMA,
            pltpu.SemaphoreType.DMA,
            pltpu.SemaphoreType.DMA,
        ],
    )
    def emb_kernel(tok_hbm, table_hbm, pos_hbm, out_hbm,
                   pos_v, idx0, idx1, gat0, gat1, buf0, buf1,
                   gsem0, gsem1, osem0, osem1, isem0, isem1):
        wid = lax.axis_index("s") * 2 + lax.axis_index("c")
        base = wid * _SEQ_PER_W
        pltpu.sync_copy(pos_hbm, pos_v)

        def start_idx(i, idx, isem):
            pltpu.async_copy(tok_hbm.at[pl.ds((base + i) * 256, 256)], idx,
                             isem)

        def drain_idx(idx, isem):
            pltpu.make_async_copy(tok_hbm.at[pl.ds(0, 256)], idx, isem).wait()

        def start_gather(idx, gat, gsem):
            pltpu.async_copy(table_hbm.at[idx.at[pl.ds(0, _C0)]],
                             gat.at[pl.ds(0, _C0)], gsem)
            pltpu.async_copy(table_hbm.at[idx.at[pl.ds(_C0, _C1)]],
                             gat.at[pl.ds(_C0, _C1)], gsem)

        def drain_gather(gat, gsem):
            pltpu.make_async_copy(table_hbm.at[pl.ds(0, _S)], gat, gsem).wait()

        start_idx(0, idx0, isem0)
        start_idx(1, idx1, isem1)
        drain_idx(idx0, isem0)
        drain_idx(idx1, isem1)
        start_gather(idx0, gat0, gsem0)
        start_gather(idx1, gat1, gsem1)

        def seq_body(j, carry):
            for s, (idx, gat, buf, gsem, osem, isem) in enumerate((
                    (idx0, gat0, buf0, gsem0, osem0, isem0),
                    (idx1, gat1, buf1, gsem1, osem1, isem1))):
                i = 2 * j + s
                b = base + i
                drain_gather(gat, gsem)

                @pl.when(i + 2 < _SEQ_PER_W)
                def _():
                    start_idx(i + 2, idx, isem)

                @pl.when(j > 0)
                def _():
                    pltpu.make_async_copy(
                        buf, out_hbm.at[pl.ds(0, _S // 2)], osem).wait()

                def add_rows(r2, c2):
                    for c in range(_D // _LANES):
                        sl = pl.ds(c * _LANES, _LANES)
                        buf[r2, sl] = (
                            gat[2 * r2, sl]
                            + pos_v[r2, pl.ds(c * _LANES, _LANES)])
                        buf[r2, pl.ds(_D + c * _LANES, _LANES)] = (
                            gat[2 * r2 + 1, sl]
                            + pos_v[r2, pl.ds(_D + c * _LANES, _LANES)])
                    return c2

                lax.fori_loop(0, _S // 2, add_rows, 0)
                pltpu.async_copy(
                    buf, out_hbm.at[pl.ds(b * (_S // 2), _S // 2)], osem)

                @pl.when(i + 2 < _SEQ_PER_W)
                def _():
                    drain_idx(idx, isem)
                    start_gather(idx, gat, gsem)
            return carry

        lax.fori_loop(0, _SEQ_PER_W // 2, seq_body, 0)
        pltpu.make_async_copy(buf0, out_hbm.at[pl.ds(0, _S // 2)],
                              osem0).wait()
        pltpu.make_async_copy(buf1, out_hbm.at[pl.ds(0, _S // 2)],
                              osem1).wait()

    return emb_kernel


_EMB_KERNEL = _make_kernel()


@jax.jit
def kernel(tokens, input_embedding, position_embedding, register_tokens):
    pos2 = position_embedding.reshape(_S // 2, 2 * _D)
    tok_flat = jnp.pad(tokens, ((0, 0), (0, 256 - _S))).reshape(-1)
    out2d = _EMB_KERNEL(tok_flat, input_embedding, pos2)
    x = out2d.reshape(_B, _S, _D)
    regs = jnp.broadcast_to(register_tokens[None], (_B, _R, _D))
    return jnp.concatenate((regs, x), axis=1)


# speedup vs baseline: 2.0264x; 2.0264x over previous
import functools

import jax
import jax.numpy as jnp
from jax import lax
from jax.experimental import pallas as pl
from jax.experimental.pallas import tpu as pltpu
from jax.experimental.pallas import tpu_sc as plsc

_B = 4096
_S = 200
_D = 64
_R = 4
_OUT_S = _R + _S
_PS = 104
_NW = 32
_SEQ_PER_W = _B // _NW
_LANES = 16
_C0 = 128
_C1 = _S - _C0


def _make_sc_kernel():
    mesh = plsc.VectorSubcoreMesh(core_axis_name="c", subcore_axis_name="s")

    @functools.partial(
        pl.kernel,
        mesh=mesh,
        compiler_params=pltpu.CompilerParams(use_tc_tiling_on_sc=False),
        out_type=jax.ShapeDtypeStruct((_B, _PS, 2 * _D), jnp.float32),
        scratch_types=[
            pltpu.VMEM((_S // 2, 2 * _D), jnp.float32),
            pltpu.VMEM((256,), jnp.int32),
            pltpu.VMEM((256,), jnp.int32),
            pltpu.VMEM((_S, _D), jnp.float32),
            pltpu.VMEM((_S, _D), jnp.float32),
            pltpu.VMEM((_PS, 2 * _D), jnp.float32),
            pltpu.VMEM((_PS, 2 * _D), jnp.float32),
            pltpu.SemaphoreType.DMA,
            pltpu.SemaphoreType.DMA,
            pltpu.SemaphoreType.DMA,
            pltpu.SemaphoreType.DMA,
            pltpu.SemaphoreType.DMA,
            pltpu.SemaphoreType.DMA,
        ],
    )
    def emb_kernel(tok_hbm, table_hbm, pos_hbm, reg_hbm, out_hbm,
                   pos_v, idx0, idx1, gat0, gat1, buf0, buf1,
                   gsem0, gsem1, osem0, osem1, isem0, isem1):
        wid = lax.axis_index("s") * 2 + lax.axis_index("c")
        base = wid * _SEQ_PER_W
        pltpu.sync_copy(pos_hbm, pos_v)
        pltpu.sync_copy(reg_hbm, buf0.at[pl.ds(0, _R // 2)])
        pltpu.sync_copy(reg_hbm, buf1.at[pl.ds(0, _R // 2)])

        def start_idx(i, idx, isem):
            pltpu.async_copy(tok_hbm.at[pl.ds((base + i) * 256, 256)], idx,
                             isem)

        def drain_idx(idx, isem):
            pltpu.make_async_copy(tok_hbm.at[pl.ds(0, 256)], idx, isem).wait()

        def start_gather(idx, gat, gsem):
            pltpu.async_copy(table_hbm.at[idx.at[pl.ds(0, _C0)]],
                             gat.at[pl.ds(0, _C0)], gsem)
            pltpu.async_copy(table_hbm.at[idx.at[pl.ds(_C0, _C1)]],
                             gat.at[pl.ds(_C0, _C1)], gsem)

        def drain_gather(gat, gsem):
            pltpu.make_async_copy(table_hbm.at[pl.ds(0, _S)], gat, gsem).wait()

        start_idx(0, idx0, isem0)
        start_idx(1, idx1, isem1)
        drain_idx(idx0, isem0)
        drain_idx(idx1, isem1)
        start_gather(idx0, gat0, gsem0)
        start_gather(idx1, gat1, gsem1)

        def seq_body(j, carry):
            for s, (idx, gat, buf, gsem, osem, isem) in enumerate((
                    (idx0, gat0, buf0, gsem0, osem0, isem0),
                    (idx1, gat1, buf1, gsem1, osem1, isem1))):
                i = 2 * j + s
                b = base + i
                drain_gather(gat, gsem)

                @pl.when(i + 2 < _SEQ_PER_W)
                def _():
                    start_idx(i + 2, idx, isem)

                @pl.when(j > 0)
                def _():
                    pltpu.make_async_copy(buf, out_hbm.at[b], osem).wait()

                def add_rows(r2, c2):
                    for c in range(_D // _LANES):
                        sl = pl.ds(c * _LANES, _LANES)
                        buf[r2 + _R // 2, sl] = (
                            gat[2 * r2, sl]
                            + pos_v[r2, pl.ds(c * _LANES, _LANES)])
                        buf[r2 + _R // 2, pl.ds(_D + c * _LANES, _LANES)] = (
                            gat[2 * r2 + 1, sl]
                            + pos_v[r2, pl.ds(_D + c * _LANES, _LANES)])
                    return c2

                lax.fori_loop(0, _S // 2, add_rows, 0)
                pltpu.async_copy(buf, out_hbm.at[b], osem)

                @pl.when(i + 2 < _SEQ_PER_W)
                def _():
                    drain_idx(idx, isem)
                    start_gather(idx, gat, gsem)
            return carry

        lax.fori_loop(0, _SEQ_PER_W // 2, seq_body, 0)
        pltpu.make_async_copy(buf0, out_hbm.at[0], osem0).wait()
        pltpu.make_async_copy(buf1, out_hbm.at[0], osem1).wait()

    return emb_kernel


_EMB_KERNEL = _make_sc_kernel()

_BBLK = 128


def _tc_transpose_body(x_ref, o_ref):
    x = x_ref[...]
    z = lax.transpose(x, (1, 2, 0))
    z = z.reshape(_PS * 2 * _D, _BBLK)
    o_ref[...] = z[: _OUT_S * _D].reshape(_OUT_S, _D, _BBLK)


_TC_TRANSPOSE = pl.pallas_call(
    _tc_transpose_body,
    grid=(_B // _BBLK,),
    in_specs=[pl.BlockSpec((_BBLK, _PS, 2 * _D), lambda i: (i, 0, 0))],
    out_specs=pl.BlockSpec((_OUT_S, _D, _BBLK), lambda i: (0, 0, i)),
    out_shape=jax.ShapeDtypeStruct((_OUT_S, _D, _B), jnp.float32),
)


@jax.jit
def kernel(tokens, input_embedding, position_embedding, register_tokens):
    pos2 = position_embedding.reshape(_S // 2, 2 * _D)
    reg2 = register_tokens.reshape(_R // 2, 2 * _D)
    tok_flat = jnp.pad(tokens, ((0, 0), (0, 256 - _S))).reshape(-1)
    staged = _EMB_KERNEL(tok_flat, input_embedding, pos2, reg2)
    out_t = _TC_TRANSPOSE(staged)
    return out_t.transpose(2, 0, 1)


# speedup vs baseline: 2.1141x; 1.0433x over previous
import functools

import jax
import jax.numpy as jnp
from jax import lax
from jax.experimental import pallas as pl
from jax.experimental.pallas import tpu as pltpu
from jax.experimental.pallas import tpu_sc as plsc

_B = 4096
_S = 200
_D = 64
_R = 4
_OUT_S = _R + _S
_PS = 104
_NW = 32
_SEQ_PER_W = _B // _NW
_LANES = 16
_C0 = 128
_C1 = _S - _C0


def _make_sc_kernel(nb):
    mesh = plsc.VectorSubcoreMesh(core_axis_name="c", subcore_axis_name="s")
    seq_per_w = nb // _NW

    @functools.partial(
        pl.kernel,
        mesh=mesh,
        compiler_params=pltpu.CompilerParams(use_tc_tiling_on_sc=False),
        out_type=jax.ShapeDtypeStruct((nb, _PS, 2 * _D), jnp.float32),
        scratch_types=[
            pltpu.VMEM((_S // 2, 2 * _D), jnp.float32),
            pltpu.VMEM((256,), jnp.int32),
            pltpu.VMEM((256,), jnp.int32),
            pltpu.VMEM((_S, _D), jnp.float32),
            pltpu.VMEM((_S, _D), jnp.float32),
            pltpu.VMEM((_PS, 2 * _D), jnp.float32),
            pltpu.VMEM((_PS, 2 * _D), jnp.float32),
            pltpu.SemaphoreType.DMA,
            pltpu.SemaphoreType.DMA,
            pltpu.SemaphoreType.DMA,
            pltpu.SemaphoreType.DMA,
            pltpu.SemaphoreType.DMA,
            pltpu.SemaphoreType.DMA,
        ],
    )
    def emb_kernel(tok_hbm, table_hbm, pos_hbm, reg_hbm, out_hbm,
                   pos_v, idx0, idx1, gat0, gat1, buf0, buf1,
                   gsem0, gsem1, osem0, osem1, isem0, isem1):
        wid = lax.axis_index("s") * 2 + lax.axis_index("c")
        base = wid * seq_per_w
        pltpu.sync_copy(pos_hbm, pos_v)
        pltpu.sync_copy(reg_hbm, buf0.at[pl.ds(0, _R // 2)])
        pltpu.sync_copy(reg_hbm, buf1.at[pl.ds(0, _R // 2)])

        def start_idx(i, idx, isem):
            pltpu.async_copy(tok_hbm.at[pl.ds((base + i) * 256, 256)], idx,
                             isem)

        def drain_idx(idx, isem):
            pltpu.make_async_copy(tok_hbm.at[pl.ds(0, 256)], idx, isem).wait()

        def start_gather(idx, gat, gsem):
            pltpu.async_copy(table_hbm.at[idx.at[pl.ds(0, _C0)]],
                             gat.at[pl.ds(0, _C0)], gsem)
            pltpu.async_copy(table_hbm.at[idx.at[pl.ds(_C0, _C1)]],
                             gat.at[pl.ds(_C0, _C1)], gsem)

        def drain_gather(gat, gsem):
            pltpu.make_async_copy(table_hbm.at[pl.ds(0, _S)], gat, gsem).wait()

        start_idx(0, idx0, isem0)
        start_idx(1, idx1, isem1)
        drain_idx(idx0, isem0)
        drain_idx(idx1, isem1)
        start_gather(idx0, gat0, gsem0)
        start_gather(idx1, gat1, gsem1)

        def seq_body(j, carry):
            for s, (idx, gat, buf, gsem, osem, isem) in enumerate((
                    (idx0, gat0, buf0, gsem0, osem0, isem0),
                    (idx1, gat1, buf1, gsem1, osem1, isem1))):
                i = 2 * j + s
                b = base + i
                drain_gather(gat, gsem)

                @pl.when(i + 2 < seq_per_w)
                def _():
                    start_idx(i + 2, idx, isem)

                @pl.when(j > 0)
                def _():
                    pltpu.make_async_copy(buf, out_hbm.at[b], osem).wait()

                def add_rows(r2, c2):
                    for c in range(_D // _LANES):
                        sl = pl.ds(c * _LANES, _LANES)
                        buf[r2 + _R // 2, sl] = (
                            gat[2 * r2, sl]
                            + pos_v[r2, pl.ds(c * _LANES, _LANES)])
                        buf[r2 + _R // 2, pl.ds(_D + c * _LANES, _LANES)] = (
                            gat[2 * r2 + 1, sl]
                            + pos_v[r2, pl.ds(_D + c * _LANES, _LANES)])
                    return c2

                lax.fori_loop(0, _S // 2, add_rows, 0)
                pltpu.async_copy(buf, out_hbm.at[b], osem)

                @pl.when(i + 2 < seq_per_w)
                def _():
                    drain_idx(idx, isem)
                    start_gather(idx, gat, gsem)
            return carry

        lax.fori_loop(0, seq_per_w // 2, seq_body, 0)
        pltpu.make_async_copy(buf0, out_hbm.at[0], osem0).wait()
        pltpu.make_async_copy(buf1, out_hbm.at[0], osem1).wait()

    return emb_kernel


_NHALF = _B // 2
_EMB_KERNEL = _make_sc_kernel(_NHALF)

_BBLK = 128
_HBLKS = _NHALF // _BBLK


def _tc_transpose_body(x_ref, o_ref):
    x = x_ref[...]
    z = lax.transpose(x, (1, 2, 0))
    z = z.reshape(_PS * 2 * _D, _BBLK)
    o_ref[...] = z[: _OUT_S * _D].reshape(_OUT_S, _D, _BBLK)


def _tc_transpose_body2(prev_ref, x_ref, o_ref):
    del prev_ref
    _tc_transpose_body(x_ref, o_ref)


_TC_TRANSPOSE_H1 = pl.pallas_call(
    _tc_transpose_body,
    grid=(_HBLKS,),
    in_specs=[pl.BlockSpec((_BBLK, _PS, 2 * _D), lambda i: (i, 0, 0))],
    out_specs=pl.BlockSpec((_OUT_S, _D, _BBLK), lambda i: (0, 0, i)),
    out_shape=jax.ShapeDtypeStruct((_OUT_S, _D, _B), jnp.float32),
)

_TC_TRANSPOSE_H2 = pl.pallas_call(
    _tc_transpose_body2,
    grid=(_HBLKS,),
    in_specs=[
        pl.BlockSpec(memory_space=pl.ANY),
        pl.BlockSpec((_BBLK, _PS, 2 * _D), lambda i: (i, 0, 0)),
    ],
    out_specs=pl.BlockSpec((_OUT_S, _D, _BBLK), lambda i: (0, 0, i + _HBLKS)),
    out_shape=jax.ShapeDtypeStruct((_OUT_S, _D, _B), jnp.float32),
    input_output_aliases={0: 0},
)


@jax.jit
def kernel(tokens, input_embedding, position_embedding, register_tokens):
    pos2 = position_embedding.reshape(_S // 2, 2 * _D)
    reg2 = register_tokens.reshape(_R // 2, 2 * _D)
    tok_flat = jnp.pad(tokens, ((0, 0), (0, 256 - _S))).reshape(-1)
    h1 = _EMB_KERNEL(tok_flat[: _NHALF * 256], input_embedding, pos2, reg2)
    h2 = _EMB_KERNEL(tok_flat[_NHALF * 256:], input_embedding, pos2, reg2)
    out_t = _TC_TRANSPOSE_H1(h1)
    out_t = _TC_TRANSPOSE_H2(out_t, h2)
    return out_t.transpose(2, 0, 1)
